# Initial kernel scaffold; baseline (speedup 1.0000x reference)
import jax
import jax.numpy as jnp
from jax.experimental import pallas as pl


def kernel(x, edge_index, edge_attr, Wq, Wk, Wv, We, Wo, bo):
    raise NotImplementedError("write your pallas kernel here")


# speedup vs baseline: 232.3274x; 232.3274x over previous
import functools

import jax
import jax.numpy as jnp
from jax import lax
from jax.experimental import pallas as pl
from jax.experimental.pallas import tpu as pltpu
from jax.experimental.pallas import tpu_sc as plsc

N_NODES = 10000
N_EDGES = 320000
DIM = 128
NUM_WORKERS = 32
EDGES_PER_WORKER = N_EDGES // NUM_WORKERS
LANES = 16


@functools.cache
def _edge_flags_kernel():
    mesh = plsc.VectorSubcoreMesh(core_axis_name="c", subcore_axis_name="s")

    @functools.partial(
        pl.kernel,
        out_type=jax.ShapeDtypeStruct((NUM_WORKERS, N_NODES), jnp.float32),
        mesh=mesh,
        scratch_types=[
            pltpu.VMEM((EDGES_PER_WORKER,), jnp.int32),
            pltpu.VMEM((N_NODES,), jnp.float32),
        ],
        compiler_params=pltpu.CompilerParams(needs_layout_passes=False),
    )
    def _edge_flags(tgt_hbm, out_hbm, idx_v, flag_v):
        wid = lax.axis_index("s") * 2 + lax.axis_index("c")
        pltpu.sync_copy(
            tgt_hbm.at[pl.ds(wid * EDGES_PER_WORKER, EDGES_PER_WORKER)], idx_v)

        zeros = jnp.zeros((LANES,), jnp.float32)

        def zero_body(i, carry):
            flag_v[pl.ds(i * LANES, LANES)] = zeros
            return carry

        lax.fori_loop(0, N_NODES // LANES, zero_body, 0)

        ones = jnp.ones((LANES,), jnp.float32)

        def scatter_body(i, carry):
            idx = idx_v[pl.ds(i * LANES, LANES)]
            plsc.store_scatter(flag_v, [idx], ones)
            return carry

        lax.fori_loop(0, EDGES_PER_WORKER // LANES, scatter_body, 0)

        pltpu.sync_copy(flag_v, out_hbm.at[wid])

    return _edge_flags


def _tc_body(x_ref, wv_ref, wo_ref, bo_ref, flags_ref, out_ref):
    deg = jnp.sum(flags_ref[...], axis=0)
    mask = (deg > 0.0).astype(jnp.float32)[:, None]
    xm = x_ref[...] * mask
    v = lax.dot_general(xm, wv_ref[...], (((1,), (1,)), ((), ())),
                        preferred_element_type=jnp.float32)
    out = lax.dot_general(v, wo_ref[...], (((1,), (1,)), ((), ())),
                          preferred_element_type=jnp.float32)
    out_ref[...] = out + bo_ref[...]


_tc_apply = pl.pallas_call(
    _tc_body,
    out_shape=jax.ShapeDtypeStruct((N_NODES, DIM), jnp.float32),
)


def kernel(x, edge_index, edge_attr, Wq, Wk, Wv, We, Wo, bo):
    tgt = edge_index[1]
    flags = _edge_flags_kernel()(tgt)
    return _tc_apply(x, Wv, Wo, bo.reshape(1, DIM), flags)


# speedup vs baseline: 324.8281x; 1.3981x over previous
import functools

import jax
import jax.numpy as jnp
from jax import lax
from jax.experimental import pallas as pl
from jax.experimental.pallas import tpu as pltpu
from jax.experimental.pallas import tpu_sc as plsc

N_NODES = 10000
N_EDGES = 320000
DIM = 128
NUM_WORKERS = 32
EDGES_PER_WORKER = N_EDGES // NUM_WORKERS
LANES = 16


@functools.cache
def _edge_flags_kernel():
    mesh = plsc.VectorSubcoreMesh(core_axis_name="c", subcore_axis_name="s")

    @functools.partial(
        pl.kernel,
        out_type=jax.ShapeDtypeStruct((NUM_WORKERS, N_NODES), jnp.float32),
        mesh=mesh,
        scratch_types=[
            pltpu.VMEM((EDGES_PER_WORKER,), jnp.int32),
            pltpu.VMEM((N_NODES,), jnp.float32),
            pltpu.SemaphoreType.DMA,
        ],
        compiler_params=pltpu.CompilerParams(needs_layout_passes=False),
    )
    def _edge_flags(tgt_hbm, out_hbm, idx_v, flag_v, sem):
        wid = lax.axis_index("s") * 2 + lax.axis_index("c")
        cp = pltpu.async_copy(
            tgt_hbm.at[pl.ds(wid * EDGES_PER_WORKER, EDGES_PER_WORKER)],
            idx_v, sem)

        zeros = jnp.zeros((LANES,), jnp.float32)

        def zero_body(i, carry):
            flag_v[pl.ds(i * LANES, LANES)] = zeros
            return carry

        lax.fori_loop(0, N_NODES // LANES, zero_body, 0)

        cp.wait()
        ones = jnp.ones((LANES,), jnp.float32)

        def scatter_body(i, carry):
            idx = idx_v[pl.ds(i * LANES, LANES)]
            plsc.store_scatter(flag_v, [idx], ones)
            return carry

        lax.fori_loop(0, EDGES_PER_WORKER // LANES, scatter_body, 0)

        pltpu.sync_copy(flag_v, out_hbm.at[wid])

    return _edge_flags


def _tgt_slice_body(ei_ref, out_ref):
    out_ref[...] = ei_ref[1, :]


_tgt_slice = pl.pallas_call(
    _tgt_slice_body,
    out_shape=jax.ShapeDtypeStruct((N_EDGES,), jnp.int32),
)


def _tc_body(x_ref, wv_ref, wo_ref, bo_ref, flags_ref, out_ref):
    deg = jnp.sum(flags_ref[...], axis=0)
    mask = (deg > 0.0).astype(jnp.float32)[:, None]
    xm = x_ref[...] * mask
    v = lax.dot_general(xm, wv_ref[...], (((1,), (1,)), ((), ())),
                        preferred_element_type=jnp.float32)
    out = lax.dot_general(v, wo_ref[...], (((1,), (1,)), ((), ())),
                          preferred_element_type=jnp.float32)
    out_ref[...] = out + bo_ref[...]


_tc_apply = pl.pallas_call(
    _tc_body,
    out_shape=jax.ShapeDtypeStruct((N_NODES, DIM), jnp.float32),
)


def kernel(x, edge_index, edge_attr, Wq, Wk, Wv, We, Wo, bo):
    tgt = _tgt_slice(edge_index)
    flags = _edge_flags_kernel()(tgt)
    return _tc_apply(x, Wv, Wo, bo.reshape(1, DIM), flags)


# speedup vs baseline: 349.9129x; 1.0772x over previous
import functools

import jax
import jax.numpy as jnp
from jax import lax
from jax.experimental import pallas as pl
from jax.experimental.pallas import tpu as pltpu
from jax.experimental.pallas import tpu_sc as plsc

N_NODES = 10000
N_EDGES = 320000
DIM = 128
NUM_WORKERS = 32
CHUNK = 10112
LANES = 16


@functools.cache
def _edge_flags_kernel():
    mesh = plsc.VectorSubcoreMesh(core_axis_name="c", subcore_axis_name="s")

    @functools.partial(
        pl.kernel,
        out_type=jax.ShapeDtypeStruct((NUM_WORKERS, N_NODES), jnp.float32),
        mesh=mesh,
        scratch_types=[
            pltpu.VMEM((2, CHUNK), jnp.int32),
            pltpu.VMEM((N_NODES,), jnp.float32),
            pltpu.SemaphoreType.DMA,
        ],
        compiler_params=pltpu.CompilerParams(needs_layout_passes=False),
    )
    def _edge_flags(ei_hbm, out_hbm, idx_v, flag_v, sem):
        wid = lax.axis_index("s") * 2 + lax.axis_index("c")
        start = jnp.minimum(wid * CHUNK, N_EDGES - CHUNK)
        cp = pltpu.async_copy(ei_hbm.at[:, pl.ds(start, CHUNK)], idx_v, sem)

        zeros = jnp.zeros((LANES,), jnp.float32)
        zunroll = 5

        def zero_body(i, carry):
            for u in range(zunroll):
                flag_v[pl.ds((i * zunroll + u) * LANES, LANES)] = zeros
            return carry

        lax.fori_loop(0, N_NODES // LANES // zunroll, zero_body, 0)

        cp.wait()
        ones = jnp.ones((LANES,), jnp.float32)
        sunroll = 8

        def scatter_body(i, carry):
            for u in range(sunroll):
                idx = idx_v[1, pl.ds((i * sunroll + u) * LANES, LANES)]
                plsc.store_scatter(flag_v, [idx], ones)
            return carry

        lax.fori_loop(0, CHUNK // LANES // sunroll, scatter_body, 0)

        pltpu.sync_copy(flag_v, out_hbm.at[wid])

    return _edge_flags


def _tc_body(x_ref, wv_ref, wo_ref, bo_ref, flags_ref, out_ref):
    deg = jnp.sum(flags_ref[...], axis=0)
    mask = (deg > 0.0).astype(jnp.float32)[:, None]
    xm = x_ref[...] * mask
    v = lax.dot_general(xm, wv_ref[...], (((1,), (1,)), ((), ())),
                        preferred_element_type=jnp.float32)
    out = lax.dot_general(v, wo_ref[...], (((1,), (1,)), ((), ())),
                          preferred_element_type=jnp.float32)
    out_ref[...] = out + bo_ref[...]


_tc_apply = pl.pallas_call(
    _tc_body,
    out_shape=jax.ShapeDtypeStruct((N_NODES, DIM), jnp.float32),
)


def kernel(x, edge_index, edge_attr, Wq, Wk, Wv, We, Wo, bo):
    flags = _edge_flags_kernel()(edge_index)
    return _tc_apply(x, Wv, Wo, bo.reshape(1, DIM), flags)
